# Initial kernel scaffold; baseline (speedup 1.0000x reference)
import jax
import jax.numpy as jnp
from jax.experimental import pallas as pl


def kernel(xyz, points, W_fc1, b_fc1, W_c1, b_c1, W_c2, b_c2, g_bn1, be_bn1, g_bn2, be_bn2, g_bn, be_bn):
    raise NotImplementedError("write your pallas kernel here")


# speedup vs baseline: 7.5139x; 7.5139x over previous
import functools

import jax
import jax.numpy as jnp
from jax import lax
from jax.experimental import pallas as pl
from jax.experimental.pallas import tpu as pltpu
from jax.experimental.pallas import tpu_sc as plsc

B = 4
N = 8192
S = 2048
K = 32
C = 64
CP = 128
EPS = 1e-5
L_TOT = B * N

NC = 2
NS = 16
NW = NC * NS
LANES = 16
RW = (B * S) // NW

F32 = jnp.float32
I32 = jnp.int32


def _fps_body(xyz_ref, idx_ref, nx_ref):
    X = xyz_ref[0, 0]
    Y = xyz_ref[0, 1]
    Z = xyz_ref[0, 2]
    row_i = lax.broadcasted_iota(I32, (64, 128), 0)
    col_i = lax.broadcasted_iota(I32, (64, 128), 1)
    flat = row_i * 128 + col_i
    srow = lax.broadcasted_iota(I32, (16, 128), 0)
    scol = lax.broadcasted_iota(I32, (16, 128), 1)
    sflat = srow * 128 + scol

    def body(i, state):
        dist, f, acc, ax, ay, az = state
        sel = sflat == i
        acc = jnp.where(sel, f, acc)
        m = flat == f
        cx = jnp.sum(jnp.where(m, X, 0.0))
        cy = jnp.sum(jnp.where(m, Y, 0.0))
        cz = jnp.sum(jnp.where(m, Z, 0.0))
        ax = jnp.where(sel, cx, ax)
        ay = jnp.where(sel, cy, ay)
        az = jnp.where(sel, cz, az)
        dx = X - cx
        dy = Y - cy
        dz = Z - cz
        d = dx * dx + dy * dy + dz * dz
        dist = jnp.minimum(dist, d)
        mx = jnp.max(dist)
        f2 = jnp.min(jnp.where(dist == mx, flat, N)).astype(I32)
        return dist, f2, acc, ax, ay, az

    dist0 = jnp.full((64, 128), 1e10, F32)
    acc0 = jnp.zeros((16, 128), I32)
    z16 = jnp.zeros((16, 128), F32)
    _, _, acc, ax, ay, az = lax.fori_loop(
        0, S, body, (dist0, jnp.int32(0), acc0, z16, z16, z16))
    idx_ref[0] = acc + pl.program_id(0) * N
    nx_ref[0, 0] = ax
    nx_ref[0, 1] = ay
    nx_ref[0, 2] = az


def _fps_call(xyz_planes):
    return pl.pallas_call(
        _fps_body,
        grid=(B,),
        in_specs=[pl.BlockSpec((1, 3, 64, 128), lambda b: (b, 0, 0, 0))],
        out_specs=[
            pl.BlockSpec((1, 16, 128), lambda b: (b, 0, 0)),
            pl.BlockSpec((1, 3, 16, 128), lambda b: (b, 0, 0, 0)),
        ],
        out_shape=[
            jax.ShapeDtypeStruct((B, 16, 128), I32),
            jax.ShapeDtypeStruct((B, 3, 16, 128), F32),
        ],
    )(xyz_planes)


RT = 2048
NT = L_TOT // RT


def _bn_scale_shift(Wm, bias, gamma, beta, s_in, m_in):
    inv_l = 1.0 / L_TOT
    ewx = lax.dot_general(s_in, Wm, (((1,), (1,)), ((), ())),
                          preferred_element_type=F32) * inv_l
    mean = ewx + bias
    wm = jnp.dot(Wm, m_in, preferred_element_type=F32)
    ey2 = jnp.sum(wm * Wm, axis=1)[None, :] * inv_l
    ey2 = ey2 + 2.0 * bias * ewx + bias * bias
    var = ey2 - mean * mean
    scale = gamma * lax.rsqrt(var + EPS)
    shift = beta - mean * scale
    return scale, shift


def _d1_body(x_ref, w_ref, b_ref, pts_ref, s1_ref, m1_ref):
    x = x_ref[...]
    p = jnp.dot(x, w_ref[...], preferred_element_type=F32) + b_ref[...]
    pts_ref[...] = jnp.concatenate([p, jnp.zeros((RT, CP - C), F32)], axis=1)

    @pl.when(pl.program_id(0) == 0)
    def _():
        s1_ref[...] = jnp.zeros_like(s1_ref)
        m1_ref[...] = jnp.zeros_like(m1_ref)

    s1_ref[...] += jnp.sum(p, axis=0, keepdims=True)
    m1_ref[...] += lax.dot_general(p, p, (((0,), (0,)), ((), ())),
                                   preferred_element_type=F32)


def _d1_call(points_flat, W_fc1, b_fc1):
    return pl.pallas_call(
        _d1_body,
        grid=(NT,),
        in_specs=[
            pl.BlockSpec((RT, C), lambda t: (t, 0)),
            pl.BlockSpec((C, C), lambda t: (0, 0)),
            pl.BlockSpec((1, C), lambda t: (0, 0)),
        ],
        out_specs=[
            pl.BlockSpec((RT, CP), lambda t: (t, 0)),
            pl.BlockSpec((1, C), lambda t: (0, 0)),
            pl.BlockSpec((C, C), lambda t: (0, 0)),
        ],
        out_shape=[
            jax.ShapeDtypeStruct((L_TOT, CP), F32),
            jax.ShapeDtypeStruct((1, C), F32),
            jax.ShapeDtypeStruct((C, C), F32),
        ],
    )(points_flat, W_fc1, b_fc1)


def _d2_body(p_ref, w1_ref, b1_ref, g1_ref, be1_ref, s1_ref, m1_ref,
             s2_ref, m2_ref):
    sc1, sh1 = _bn_scale_shift(w1_ref[...], b1_ref[...], g1_ref[...],
                               be1_ref[...], s1_ref[...], m1_ref[...])
    p = p_ref[:, :C]
    y1 = lax.dot_general(p, w1_ref[...], (((1,), (1,)), ((), ())),
                         preferred_element_type=F32)
    h1 = jnp.maximum(y1 * sc1 + (b1_ref[...] * sc1 + sh1), 0.0)

    @pl.when(pl.program_id(0) == 0)
    def _():
        s2_ref[...] = jnp.zeros_like(s2_ref)
        m2_ref[...] = jnp.zeros_like(m2_ref)

    s2_ref[...] += jnp.sum(h1, axis=0, keepdims=True)
    m2_ref[...] += lax.dot_general(h1, h1, (((0,), (0,)), ((), ())),
                                   preferred_element_type=F32)


def _d2_call(pts, W_c1, b_c1, g1, be1, S1, M1):
    return pl.pallas_call(
        _d2_body,
        grid=(NT,),
        in_specs=[
            pl.BlockSpec((RT, CP), lambda t: (t, 0)),
            pl.BlockSpec((C, C), lambda t: (0, 0)),
            pl.BlockSpec((1, C), lambda t: (0, 0)),
            pl.BlockSpec((1, C), lambda t: (0, 0)),
            pl.BlockSpec((1, C), lambda t: (0, 0)),
            pl.BlockSpec((1, C), lambda t: (0, 0)),
            pl.BlockSpec((C, C), lambda t: (0, 0)),
        ],
        out_specs=[
            pl.BlockSpec((1, C), lambda t: (0, 0)),
            pl.BlockSpec((C, C), lambda t: (0, 0)),
        ],
        out_shape=[
            jax.ShapeDtypeStruct((1, C), F32),
            jax.ShapeDtypeStruct((C, C), F32),
        ],
    )(pts, W_c1, b_c1, g1, be1, S1, M1)


def _d3_body(p_ref, w1_ref, b1_ref, g1_ref, be1_ref, w2_ref, b2_ref, g2_ref,
             be2_ref, s1_ref, m1_ref, s2_ref, m2_ref, out_ref):
    sc1, sh1 = _bn_scale_shift(w1_ref[...], b1_ref[...], g1_ref[...],
                               be1_ref[...], s1_ref[...], m1_ref[...])
    sc2, sh2 = _bn_scale_shift(w2_ref[...], b2_ref[...], g2_ref[...],
                               be2_ref[...], s2_ref[...], m2_ref[...])
    p = p_ref[:, :C]
    y1 = lax.dot_general(p, w1_ref[...], (((1,), (1,)), ((), ())),
                         preferred_element_type=F32)
    h1 = jnp.maximum(y1 * sc1 + (b1_ref[...] * sc1 + sh1), 0.0)
    y2 = lax.dot_general(h1, w2_ref[...], (((1,), (1,)), ((), ())),
                         preferred_element_type=F32)
    h2 = jnp.maximum(y2 * sc2 + (b2_ref[...] * sc2 + sh2), 0.0)
    out_ref[...] = jnp.concatenate([p + h2, jnp.zeros((RT, CP - C), F32)],
                                   axis=1)


def _d3_call(pts, W_c1, b_c1, g1, be1, W_c2, b_c2, g2, be2, S1, M1, S2, M2):
    vec = pl.BlockSpec((1, C), lambda t: (0, 0))
    mat = pl.BlockSpec((C, C), lambda t: (0, 0))
    return pl.pallas_call(
        _d3_body,
        grid=(NT,),
        in_specs=[pl.BlockSpec((RT, CP), lambda t: (t, 0)),
                  mat, vec, vec, vec, mat, vec, vec, vec, vec, mat, vec, mat],
        out_specs=pl.BlockSpec((RT, CP), lambda t: (t, 0)),
        out_shape=jax.ShapeDtypeStruct((L_TOT, CP), F32),
    )(pts, W_c1, b_c1, g1, be1, W_c2, b_c2, g2, be2, S1, M1, S2, M2)


RS = 256
NQT = S // RS


def _knn_body(nx_ref, xp_ref, kidx_ref, d_ref):
    q = nx_ref[0]
    x = xp_ref[0]
    t = lax.dot_general(q, x, (((0,), (0,)), ((), ())),
                        preferred_element_type=F32)
    qsq = jnp.sum(q * q, axis=0)[:, None]
    xsq = jnp.sum(x * x, axis=0)[None, :]
    d_ref[...] = (-2.0 * t + qsq) + xsq

    col = lax.broadcasted_iota(I32, (RS, N), 1)
    kcol = lax.broadcasted_iota(I32, (RS, K), 1)
    big = jnp.float32(3.0e38)

    def step(k, acc):
        d = d_ref[...]
        m = jnp.min(d, axis=1, keepdims=True)
        j = jnp.min(jnp.where(d == m, col, N), axis=1, keepdims=True)
        acc = jnp.where(kcol == k, j, acc)
        d_ref[...] = jnp.where(col == j, big, d)
        return acc

    acc = lax.fori_loop(0, K, step, jnp.zeros((RS, K), I32))
    kidx_ref[0] = acc + pl.program_id(0) * N


def _knn_call(nx_planes, xyz_planes):
    return pl.pallas_call(
        _knn_body,
        grid=(B, NQT),
        in_specs=[
            pl.BlockSpec((1, 3, RS), lambda b, t: (b, 0, t)),
            pl.BlockSpec((1, 3, N), lambda b, t: (b, 0, 0)),
        ],
        out_specs=pl.BlockSpec((1, RS, K), lambda b, t: (b, t, 0)),
        out_shape=jax.ShapeDtypeStruct((B, S, K), I32),
        scratch_shapes=[pltpu.VMEM((RS, N), F32)],
    )(nx_planes, xyz_planes)


def _sca_body(pts_hbm, gidx_hbm, pori_hbm, gidx_v, po_v, sem):
    wid = lax.axis_index("s") * NC + lax.axis_index("c")
    base = wid * RW
    pltpu.sync_copy(gidx_hbm.at[pl.ds(base, RW)], gidx_v)
    pltpu.async_copy(pts_hbm.at[gidx_v], po_v, sem).wait()
    pltpu.sync_copy(po_v, pori_hbm.at[pl.ds(base, RW)])


def _scb_body(pts2_hbm, kidx_hbm, maxp_hbm, kidx_v, grp0_v, grp1_v, res_v,
              sem):
    wid = lax.axis_index("s") * NC + lax.axis_index("c")
    base = wid * RW

    pltpu.sync_copy(kidx_hbm.at[pl.ds(base * K, RW * K)], kidx_v)

    def start(r, grp):
        pltpu.make_async_copy(
            pts2_hbm.at[kidx_v.at[pl.ds(r * K, K)]], grp, sem).start()

    def wait(grp):
        pltpu.make_async_copy(
            pts2_hbm.at[kidx_v.at[pl.ds(0, K)]], grp, sem).wait()

    def compute(r, grp):
        for j in range(C // LANES):
            sl = pl.ds(j * LANES, LANES)
            a = grp[0, sl]
            for k in range(1, K):
                a = jnp.maximum(a, grp[k, sl])
            res_v[pl.ds(r * C + j * LANES, LANES)] = a

    start(0, grp0_v)
    start(1, grp1_v)

    def body(r2, carry):
        r = r2 * 2
        wait(grp0_v)
        compute(r, grp0_v)

        @pl.when(r + 2 < RW)
        def _():
            start(r + 2, grp0_v)

        wait(grp1_v)
        compute(r + 1, grp1_v)

        @pl.when(r + 3 < RW)
        def _():
            start(r + 3, grp1_v)

        return carry

    lax.fori_loop(0, RW // 2, body, 0)
    pltpu.sync_copy(res_v, maxp_hbm.at[pl.ds(base * C, RW * C)])


@functools.lru_cache(maxsize=None)
def _get_sc_kernels():
    mesh = plsc.VectorSubcoreMesh(core_axis_name="c", subcore_axis_name="s",
                                  num_cores=NC, num_subcores=NS)
    sca = functools.partial(
        pl.kernel,
        out_type=jax.ShapeDtypeStruct((B * S, CP), F32),
        mesh=mesh,
        scratch_types=[
            pltpu.VMEM((RW,), I32),
            pltpu.VMEM((RW, CP), F32),
            pltpu.SemaphoreType.DMA,
        ],
    )(_sca_body)
    scb = functools.partial(
        pl.kernel,
        out_type=jax.ShapeDtypeStruct((B * S * C,), F32),
        mesh=mesh,
        scratch_types=[
            pltpu.VMEM((RW * K,), I32),
            pltpu.VMEM((K, CP), F32),
            pltpu.VMEM((K, CP), F32),
            pltpu.VMEM((RW * C,), F32),
            pltpu.SemaphoreType.DMA,
        ],
    )(_scb_body)
    return sca, scb


FT = 2048
NFT = (B * S) // FT


def _fbn_stats_body(mx_ref, po_ref, np_ref, s_ref, q_ref):
    v = mx_ref[...] + po_ref[:, :C]
    np_ref[...] = v

    @pl.when(pl.program_id(0) == 0)
    def _():
        s_ref[...] = jnp.zeros_like(s_ref)
        q_ref[...] = jnp.zeros_like(q_ref)

    s_ref[...] += jnp.sum(v, axis=0, keepdims=True)
    q_ref[...] += jnp.sum(v * v, axis=0, keepdims=True)


def _fbn_stats_call(maxp, pori):
    return pl.pallas_call(
        _fbn_stats_body,
        grid=(NFT,),
        in_specs=[
            pl.BlockSpec((FT, C), lambda t: (t, 0)),
            pl.BlockSpec((FT, CP), lambda t: (t, 0)),
        ],
        out_specs=[
            pl.BlockSpec((FT, C), lambda t: (t, 0)),
            pl.BlockSpec((1, C), lambda t: (0, 0)),
            pl.BlockSpec((1, C), lambda t: (0, 0)),
        ],
        out_shape=[
            jax.ShapeDtypeStruct((B * S, C), F32),
            jax.ShapeDtypeStruct((1, C), F32),
            jax.ShapeDtypeStruct((1, C), F32),
        ],
    )(maxp, pori)


def _fbn_norm_body(np_ref, s_ref, q_ref, g_ref, be_ref, out_ref):
    inv_l = 1.0 / (B * S)
    mean = s_ref[...] * inv_l
    var = q_ref[...] * inv_l - mean * mean
    scale = g_ref[...] * lax.rsqrt(var + EPS)
    shift = be_ref[...] - mean * scale
    out_ref[...] = np_ref[...] * scale + shift


def _fbn_norm_call(newp, ssum, qsum, g_bn, be_bn):
    vec = pl.BlockSpec((1, C), lambda t: (0, 0))
    return pl.pallas_call(
        _fbn_norm_body,
        grid=(NFT,),
        in_specs=[pl.BlockSpec((FT, C), lambda t: (t, 0)), vec, vec, vec, vec],
        out_specs=pl.BlockSpec((FT, C), lambda t: (t, 0)),
        out_shape=jax.ShapeDtypeStruct((B * S, C), F32),
    )(newp, ssum, qsum, g_bn, be_bn)


def kernel(xyz, points, W_fc1, b_fc1, W_c1, b_c1, W_c2, b_c2,
           g_bn1, be_bn1, g_bn2, be_bn2, g_bn, be_bn):
    xyzp = xyz.transpose(0, 2, 1)
    xyzp4 = xyzp.reshape(B, 3, 64, 128)

    b_fc1r = b_fc1.reshape(1, C)
    b1 = b_c1.reshape(1, C)
    b2 = b_c2.reshape(1, C)
    g1 = g_bn1.reshape(1, C)
    be1 = be_bn1.reshape(1, C)
    g2 = g_bn2.reshape(1, C)
    be2 = be_bn2.reshape(1, C)
    gf = g_bn.reshape(1, C)
    bef = be_bn.reshape(1, C)

    gidx4, nxp4 = _fps_call(xyzp4)
    gidx = gidx4.reshape(B * S)
    nx_planes = nxp4.reshape(B, 3, S)
    new_xyz = nx_planes.transpose(0, 2, 1)

    pts, S1, M1 = _d1_call(points.reshape(L_TOT, C), W_fc1, b_fc1r)

    sca, scb = _get_sc_kernels()
    pori = sca(pts, gidx)

    S2, M2 = _d2_call(pts, W_c1, b1, g1, be1, S1, M1)
    pts2 = _d3_call(pts, W_c1, b1, g1, be1, W_c2, b2, g2, be2, S1, M1, S2, M2)

    kidx = _knn_call(nx_planes, xyzp).reshape(B * S * K)

    maxp = scb(pts2, kidx).reshape(B * S, C)

    newp, ssum, qsum = _fbn_stats_call(maxp, pori)
    new_points = _fbn_norm_call(newp, ssum, qsum, gf, bef).reshape(B, S, C)

    return (new_xyz, new_points)


# speedup vs baseline: 7.8311x; 1.0422x over previous
import functools

import jax
import jax.numpy as jnp
from jax import lax
from jax.experimental import pallas as pl
from jax.experimental.pallas import tpu as pltpu
from jax.experimental.pallas import tpu_sc as plsc

B = 4
N = 8192
S = 2048
K = 32
C = 64
CP = 128
EPS = 1e-5
L_TOT = B * N

NC = 2
NS = 16
NW = NC * NS
LANES = 16
RW = (B * S) // NW

F32 = jnp.float32
I32 = jnp.int32


def _fps_body(xyz_ref, xyzr_ref, idx_ref, nx_ref):
    X = xyz_ref[0, 0]
    Y = xyz_ref[0, 1]
    Z = xyz_ref[0, 2]
    row_i = lax.broadcasted_iota(I32, (64, 128), 0)
    col_i = lax.broadcasted_iota(I32, (64, 128), 1)
    flat = row_i * 128 + col_i
    srow = lax.broadcasted_iota(I32, (16, 128), 0)
    scol = lax.broadcasted_iota(I32, (16, 128), 1)
    sflat = srow * 128 + scol

    def body(i, state):
        dist, f, acc, ax, ay, az = state
        sel = sflat == i
        acc = jnp.where(sel, f, acc)
        cen = xyzr_ref[0, pl.ds(f, 1), :]
        cx = cen[:, 0:1]
        cy = cen[:, 1:2]
        cz = cen[:, 2:3]
        ax = jnp.where(sel, cx, ax)
        ay = jnp.where(sel, cy, ay)
        az = jnp.where(sel, cz, az)
        dx = X - cx
        dy = Y - cy
        dz = Z - cz
        d = dx * dx + dy * dy + dz * dz
        dist = jnp.minimum(dist, d)
        mx = jnp.max(dist)
        f2 = jnp.min(jnp.where(dist == mx, flat, N)).astype(I32)
        return dist, f2, acc, ax, ay, az

    dist0 = jnp.full((64, 128), 1e10, F32)
    acc0 = jnp.zeros((16, 128), I32)
    z16 = jnp.zeros((16, 128), F32)
    _, _, acc, ax, ay, az = lax.fori_loop(
        0, S, body, (dist0, jnp.int32(0), acc0, z16, z16, z16))
    idx_ref[0] = acc + pl.program_id(0) * N
    nx_ref[0, 0] = ax
    nx_ref[0, 1] = ay
    nx_ref[0, 2] = az


def _fps_call(xyz_planes, xyz_rows):
    return pl.pallas_call(
        _fps_body,
        grid=(B,),
        in_specs=[pl.BlockSpec((1, 3, 64, 128), lambda b: (b, 0, 0, 0)),
                  pl.BlockSpec((1, N, 3), lambda b: (b, 0, 0))],
        out_specs=[
            pl.BlockSpec((1, 16, 128), lambda b: (b, 0, 0)),
            pl.BlockSpec((1, 3, 16, 128), lambda b: (b, 0, 0, 0)),
        ],
        out_shape=[
            jax.ShapeDtypeStruct((B, 16, 128), I32),
            jax.ShapeDtypeStruct((B, 3, 16, 128), F32),
        ],
    )(xyz_planes, xyz_rows)


RT = 2048
NT = L_TOT // RT


def _bn_scale_shift(Wm, bias, gamma, beta, s_in, m_in):
    inv_l = 1.0 / L_TOT
    ewx = lax.dot_general(s_in, Wm, (((1,), (1,)), ((), ())),
                          preferred_element_type=F32) * inv_l
    mean = ewx + bias
    wm = jnp.dot(Wm, m_in, preferred_element_type=F32)
    ey2 = jnp.sum(wm * Wm, axis=1)[None, :] * inv_l
    ey2 = ey2 + 2.0 * bias * ewx + bias * bias
    var = ey2 - mean * mean
    scale = gamma * lax.rsqrt(var + EPS)
    shift = beta - mean * scale
    return scale, shift


def _d1_body(x_ref, w_ref, b_ref, pts_ref, s1_ref, m1_ref):
    x = x_ref[...]
    p = jnp.dot(x, w_ref[...], preferred_element_type=F32) + b_ref[...]
    pts_ref[...] = jnp.concatenate([p, jnp.zeros((RT, CP - C), F32)], axis=1)

    @pl.when(pl.program_id(0) == 0)
    def _():
        s1_ref[...] = jnp.zeros_like(s1_ref)
        m1_ref[...] = jnp.zeros_like(m1_ref)

    s1_ref[...] += jnp.sum(p, axis=0, keepdims=True)
    m1_ref[...] += lax.dot_general(p, p, (((0,), (0,)), ((), ())),
                                   preferred_element_type=F32)


def _d1_call(points_flat, W_fc1, b_fc1):
    return pl.pallas_call(
        _d1_body,
        grid=(NT,),
        in_specs=[
            pl.BlockSpec((RT, C), lambda t: (t, 0)),
            pl.BlockSpec((C, C), lambda t: (0, 0)),
            pl.BlockSpec((1, C), lambda t: (0, 0)),
        ],
        out_specs=[
            pl.BlockSpec((RT, CP), lambda t: (t, 0)),
            pl.BlockSpec((1, C), lambda t: (0, 0)),
            pl.BlockSpec((C, C), lambda t: (0, 0)),
        ],
        out_shape=[
            jax.ShapeDtypeStruct((L_TOT, CP), F32),
            jax.ShapeDtypeStruct((1, C), F32),
            jax.ShapeDtypeStruct((C, C), F32),
        ],
    )(points_flat, W_fc1, b_fc1)


def _d2_body(p_ref, w1_ref, b1_ref, g1_ref, be1_ref, s1_ref, m1_ref,
             s2_ref, m2_ref):
    sc1, sh1 = _bn_scale_shift(w1_ref[...], b1_ref[...], g1_ref[...],
                               be1_ref[...], s1_ref[...], m1_ref[...])
    p = p_ref[:, :C]
    y1 = lax.dot_general(p, w1_ref[...], (((1,), (1,)), ((), ())),
                         preferred_element_type=F32)
    h1 = jnp.maximum(y1 * sc1 + (b1_ref[...] * sc1 + sh1), 0.0)

    @pl.when(pl.program_id(0) == 0)
    def _():
        s2_ref[...] = jnp.zeros_like(s2_ref)
        m2_ref[...] = jnp.zeros_like(m2_ref)

    s2_ref[...] += jnp.sum(h1, axis=0, keepdims=True)
    m2_ref[...] += lax.dot_general(h1, h1, (((0,), (0,)), ((), ())),
                                   preferred_element_type=F32)


def _d2_call(pts, W_c1, b_c1, g1, be1, S1, M1):
    return pl.pallas_call(
        _d2_body,
        grid=(NT,),
        in_specs=[
            pl.BlockSpec((RT, CP), lambda t: (t, 0)),
            pl.BlockSpec((C, C), lambda t: (0, 0)),
            pl.BlockSpec((1, C), lambda t: (0, 0)),
            pl.BlockSpec((1, C), lambda t: (0, 0)),
            pl.BlockSpec((1, C), lambda t: (0, 0)),
            pl.BlockSpec((1, C), lambda t: (0, 0)),
            pl.BlockSpec((C, C), lambda t: (0, 0)),
        ],
        out_specs=[
            pl.BlockSpec((1, C), lambda t: (0, 0)),
            pl.BlockSpec((C, C), lambda t: (0, 0)),
        ],
        out_shape=[
            jax.ShapeDtypeStruct((1, C), F32),
            jax.ShapeDtypeStruct((C, C), F32),
        ],
    )(pts, W_c1, b_c1, g1, be1, S1, M1)


def _d3_body(p_ref, w1_ref, b1_ref, g1_ref, be1_ref, w2_ref, b2_ref, g2_ref,
             be2_ref, s1_ref, m1_ref, s2_ref, m2_ref, out_ref):
    sc1, sh1 = _bn_scale_shift(w1_ref[...], b1_ref[...], g1_ref[...],
                               be1_ref[...], s1_ref[...], m1_ref[...])
    sc2, sh2 = _bn_scale_shift(w2_ref[...], b2_ref[...], g2_ref[...],
                               be2_ref[...], s2_ref[...], m2_ref[...])
    p = p_ref[:, :C]
    y1 = lax.dot_general(p, w1_ref[...], (((1,), (1,)), ((), ())),
                         preferred_element_type=F32)
    h1 = jnp.maximum(y1 * sc1 + (b1_ref[...] * sc1 + sh1), 0.0)
    y2 = lax.dot_general(h1, w2_ref[...], (((1,), (1,)), ((), ())),
                         preferred_element_type=F32)
    h2 = jnp.maximum(y2 * sc2 + (b2_ref[...] * sc2 + sh2), 0.0)
    out_ref[...] = jnp.concatenate([p + h2, jnp.zeros((RT, CP - C), F32)],
                                   axis=1)


def _d3_call(pts, W_c1, b_c1, g1, be1, W_c2, b_c2, g2, be2, S1, M1, S2, M2):
    vec = pl.BlockSpec((1, C), lambda t: (0, 0))
    mat = pl.BlockSpec((C, C), lambda t: (0, 0))
    return pl.pallas_call(
        _d3_body,
        grid=(NT,),
        in_specs=[pl.BlockSpec((RT, CP), lambda t: (t, 0)),
                  mat, vec, vec, vec, mat, vec, vec, vec, vec, mat, vec, mat],
        out_specs=pl.BlockSpec((RT, CP), lambda t: (t, 0)),
        out_shape=jax.ShapeDtypeStruct((L_TOT, CP), F32),
    )(pts, W_c1, b_c1, g1, be1, W_c2, b_c2, g2, be2, S1, M1, S2, M2)


RS = 256
NQT = S // RS


def _knn_body(nx_ref, xp_ref, kidx_ref, d_ref):
    q = nx_ref[0]
    x = xp_ref[0]
    t = lax.dot_general(q, x, (((0,), (0,)), ((), ())),
                        preferred_element_type=F32)
    qsq = jnp.sum(q * q, axis=0)[:, None]
    xsq = jnp.sum(x * x, axis=0)[None, :]
    d_ref[...] = (-2.0 * t + qsq) + xsq

    col = lax.broadcasted_iota(I32, (RS, N), 1)
    kcol = lax.broadcasted_iota(I32, (RS, K), 1)
    big = jnp.float32(3.0e38)

    def step(k, acc):
        d = d_ref[...]
        m = jnp.min(d, axis=1, keepdims=True)
        eq = d == m
        j = jnp.min(jnp.where(eq, col, N), axis=1, keepdims=True)
        acc = jnp.where(kcol == k, j, acc)
        d_ref[...] = jnp.where(eq, big, d)
        return acc

    acc = lax.fori_loop(0, K, step, jnp.zeros((RS, K), I32))
    kidx_ref[0] = acc + pl.program_id(0) * N


def _knn_call(nx_planes, xyz_planes):
    return pl.pallas_call(
        _knn_body,
        grid=(B, NQT),
        in_specs=[
            pl.BlockSpec((1, 3, RS), lambda b, t: (b, 0, t)),
            pl.BlockSpec((1, 3, N), lambda b, t: (b, 0, 0)),
        ],
        out_specs=pl.BlockSpec((1, RS, K), lambda b, t: (b, t, 0)),
        out_shape=jax.ShapeDtypeStruct((B, S, K), I32),
        scratch_shapes=[pltpu.VMEM((RS, N), F32)],
    )(nx_planes, xyz_planes)


def _sca_body(pts_hbm, gidx_hbm, pori_hbm, gidx_v, po_v, sem):
    wid = lax.axis_index("s") * NC + lax.axis_index("c")
    base = wid * RW
    pltpu.sync_copy(gidx_hbm.at[pl.ds(base, RW)], gidx_v)
    pltpu.async_copy(pts_hbm.at[gidx_v], po_v, sem).wait()
    pltpu.sync_copy(po_v, pori_hbm.at[pl.ds(base, RW)])


def _scb_body(pts2_hbm, kidx_hbm, maxp_hbm, kidx_v, grp0_v, grp1_v, res_v,
              sem):
    wid = lax.axis_index("s") * NC + lax.axis_index("c")
    base = wid * RW

    pltpu.sync_copy(kidx_hbm.at[pl.ds(base * K, RW * K)], kidx_v)

    def start(r, grp):
        pltpu.make_async_copy(
            pts2_hbm.at[kidx_v.at[pl.ds(r * K, K)]], grp, sem).start()

    def wait(grp):
        pltpu.make_async_copy(
            pts2_hbm.at[kidx_v.at[pl.ds(0, K)]], grp, sem).wait()

    def compute(r, grp):
        for j in range(C // LANES):
            sl = pl.ds(j * LANES, LANES)
            a = grp[0, sl]
            for k in range(1, K):
                a = jnp.maximum(a, grp[k, sl])
            res_v[pl.ds(r * C + j * LANES, LANES)] = a

    start(0, grp0_v)
    start(1, grp1_v)

    def body(r2, carry):
        r = r2 * 2
        wait(grp0_v)
        compute(r, grp0_v)

        @pl.when(r + 2 < RW)
        def _():
            start(r + 2, grp0_v)

        wait(grp1_v)
        compute(r + 1, grp1_v)

        @pl.when(r + 3 < RW)
        def _():
            start(r + 3, grp1_v)

        return carry

    lax.fori_loop(0, RW // 2, body, 0)
    pltpu.sync_copy(res_v, maxp_hbm.at[pl.ds(base * C, RW * C)])


@functools.lru_cache(maxsize=None)
def _get_sc_kernels():
    mesh = plsc.VectorSubcoreMesh(core_axis_name="c", subcore_axis_name="s",
                                  num_cores=NC, num_subcores=NS)
    sca = functools.partial(
        pl.kernel,
        out_type=jax.ShapeDtypeStruct((B * S, CP), F32),
        mesh=mesh,
        scratch_types=[
            pltpu.VMEM((RW,), I32),
            pltpu.VMEM((RW, CP), F32),
            pltpu.SemaphoreType.DMA,
        ],
    )(_sca_body)
    scb = functools.partial(
        pl.kernel,
        out_type=jax.ShapeDtypeStruct((B * S * C,), F32),
        mesh=mesh,
        scratch_types=[
            pltpu.VMEM((RW * K,), I32),
            pltpu.VMEM((K, CP), F32),
            pltpu.VMEM((K, CP), F32),
            pltpu.VMEM((RW * C,), F32),
            pltpu.SemaphoreType.DMA,
        ],
    )(_scb_body)
    return sca, scb


FT = 2048
NFT = (B * S) // FT


def _fbn_stats_body(mx_ref, po_ref, np_ref, s_ref, q_ref):
    v = mx_ref[...] + po_ref[:, :C]
    np_ref[...] = v

    @pl.when(pl.program_id(0) == 0)
    def _():
        s_ref[...] = jnp.zeros_like(s_ref)
        q_ref[...] = jnp.zeros_like(q_ref)

    s_ref[...] += jnp.sum(v, axis=0, keepdims=True)
    q_ref[...] += jnp.sum(v * v, axis=0, keepdims=True)


def _fbn_stats_call(maxp, pori):
    return pl.pallas_call(
        _fbn_stats_body,
        grid=(NFT,),
        in_specs=[
            pl.BlockSpec((FT, C), lambda t: (t, 0)),
            pl.BlockSpec((FT, CP), lambda t: (t, 0)),
        ],
        out_specs=[
            pl.BlockSpec((FT, C), lambda t: (t, 0)),
            pl.BlockSpec((1, C), lambda t: (0, 0)),
            pl.BlockSpec((1, C), lambda t: (0, 0)),
        ],
        out_shape=[
            jax.ShapeDtypeStruct((B * S, C), F32),
            jax.ShapeDtypeStruct((1, C), F32),
            jax.ShapeDtypeStruct((1, C), F32),
        ],
    )(maxp, pori)


def _fbn_norm_body(np_ref, s_ref, q_ref, g_ref, be_ref, out_ref):
    inv_l = 1.0 / (B * S)
    mean = s_ref[...] * inv_l
    var = q_ref[...] * inv_l - mean * mean
    scale = g_ref[...] * lax.rsqrt(var + EPS)
    shift = be_ref[...] - mean * scale
    out_ref[...] = np_ref[...] * scale + shift


def _fbn_norm_call(newp, ssum, qsum, g_bn, be_bn):
    vec = pl.BlockSpec((1, C), lambda t: (0, 0))
    return pl.pallas_call(
        _fbn_norm_body,
        grid=(NFT,),
        in_specs=[pl.BlockSpec((FT, C), lambda t: (t, 0)), vec, vec, vec, vec],
        out_specs=pl.BlockSpec((FT, C), lambda t: (t, 0)),
        out_shape=jax.ShapeDtypeStruct((B * S, C), F32),
    )(newp, ssum, qsum, g_bn, be_bn)


def kernel(xyz, points, W_fc1, b_fc1, W_c1, b_c1, W_c2, b_c2,
           g_bn1, be_bn1, g_bn2, be_bn2, g_bn, be_bn):
    xyzp = xyz.transpose(0, 2, 1)
    xyzp4 = xyzp.reshape(B, 3, 64, 128)

    b_fc1r = b_fc1.reshape(1, C)
    b1 = b_c1.reshape(1, C)
    b2 = b_c2.reshape(1, C)
    g1 = g_bn1.reshape(1, C)
    be1 = be_bn1.reshape(1, C)
    g2 = g_bn2.reshape(1, C)
    be2 = be_bn2.reshape(1, C)
    gf = g_bn.reshape(1, C)
    bef = be_bn.reshape(1, C)

    gidx4, nxp4 = _fps_call(xyzp4, xyz)
    gidx = gidx4.reshape(B * S)
    nx_planes = nxp4.reshape(B, 3, S)
    new_xyz = nx_planes.transpose(0, 2, 1)

    pts, S1, M1 = _d1_call(points.reshape(L_TOT, C), W_fc1, b_fc1r)

    sca, scb = _get_sc_kernels()
    pori = sca(pts, gidx)

    S2, M2 = _d2_call(pts, W_c1, b1, g1, be1, S1, M1)
    pts2 = _d3_call(pts, W_c1, b1, g1, be1, W_c2, b2, g2, be2, S1, M1, S2, M2)

    kidx = _knn_call(nx_planes, xyzp).reshape(B * S * K)

    maxp = scb(pts2, kidx).reshape(B * S, C)

    newp, ssum, qsum = _fbn_stats_call(maxp, pori)
    new_points = _fbn_norm_call(newp, ssum, qsum, gf, bef).reshape(B, S, C)

    return (new_xyz, new_points)


# speedup vs baseline: 8.6231x; 1.1011x over previous
import functools

import jax
import jax.numpy as jnp
from jax import lax
from jax.experimental import pallas as pl
from jax.experimental.pallas import tpu as pltpu
from jax.experimental.pallas import tpu_sc as plsc

B = 4
N = 8192
S = 2048
K = 32
C = 64
CP = 128
EPS = 1e-5
L_TOT = B * N

NC = 2
NS = 16
NW = NC * NS
LANES = 16
RW = (B * S) // NW

F32 = jnp.float32
I32 = jnp.int32


def _fps_body(xyz_ref, xyzr_ref, idx_ref, nx_ref):
    Xs = [xyz_ref[b, 0] for b in range(B)]
    Ys = [xyz_ref[b, 1] for b in range(B)]
    Zs = [xyz_ref[b, 2] for b in range(B)]
    row_i = lax.broadcasted_iota(I32, (64, 128), 0)
    col_i = lax.broadcasted_iota(I32, (64, 128), 1)
    flat = row_i * 128 + col_i
    srow = lax.broadcasted_iota(I32, (16, 128), 0)
    scol = lax.broadcasted_iota(I32, (16, 128), 1)
    sflat = srow * 128 + scol

    def body(i, state):
        dists, fs, accs, axs, ays, azs = state
        sel = sflat == i
        new = [(), (), (), (), (), ()]
        for b in range(B):
            acc = jnp.where(sel, fs[b], accs[b])
            cen = xyzr_ref[b, pl.ds(fs[b], 1), :]
            cx = cen[:, 0:1]
            cy = cen[:, 1:2]
            cz = cen[:, 2:3]
            ax = jnp.where(sel, cx, axs[b])
            ay = jnp.where(sel, cy, ays[b])
            az = jnp.where(sel, cz, azs[b])
            dx = Xs[b] - cx
            dy = Ys[b] - cy
            dz = Zs[b] - cz
            d = dx * dx + dy * dy + dz * dz
            dist = jnp.minimum(dists[b], d)
            mx = jnp.max(dist)
            f2 = jnp.min(jnp.where(dist == mx, flat, N)).astype(I32)
            new[0] += (dist,)
            new[1] += (f2,)
            new[2] += (acc,)
            new[3] += (ax,)
            new[4] += (ay,)
            new[5] += (az,)
        return tuple(new)

    dist0 = jnp.full((64, 128), 1e10, F32)
    acc0 = jnp.zeros((16, 128), I32)
    z16 = jnp.zeros((16, 128), F32)
    state0 = ((dist0,) * B, (jnp.int32(0),) * B, (acc0,) * B,
              (z16,) * B, (z16,) * B, (z16,) * B)
    _, _, accs, axs, ays, azs = lax.fori_loop(0, S, body, state0)
    for b in range(B):
        idx_ref[b] = accs[b] + b * N
        nx_ref[b, 0] = axs[b]
        nx_ref[b, 1] = ays[b]
        nx_ref[b, 2] = azs[b]


def _fps_call(xyz_planes, xyz_rows):
    return pl.pallas_call(
        _fps_body,
        in_specs=[pl.BlockSpec((B, 3, 64, 128), lambda: (0, 0, 0, 0)),
                  pl.BlockSpec((B, N, 3), lambda: (0, 0, 0))],
        out_specs=[
            pl.BlockSpec((B, 16, 128), lambda: (0, 0, 0)),
            pl.BlockSpec((B, 3, 16, 128), lambda: (0, 0, 0, 0)),
        ],
        out_shape=[
            jax.ShapeDtypeStruct((B, 16, 128), I32),
            jax.ShapeDtypeStruct((B, 3, 16, 128), F32),
        ],
    )(xyz_planes, xyz_rows)


RT = 2048
NT = L_TOT // RT


def _bn_scale_shift(Wm, bias, gamma, beta, s_in, m_in):
    inv_l = 1.0 / L_TOT
    ewx = lax.dot_general(s_in, Wm, (((1,), (1,)), ((), ())),
                          preferred_element_type=F32) * inv_l
    mean = ewx + bias
    wm = jnp.dot(Wm, m_in, preferred_element_type=F32)
    ey2 = jnp.sum(wm * Wm, axis=1)[None, :] * inv_l
    ey2 = ey2 + 2.0 * bias * ewx + bias * bias
    var = ey2 - mean * mean
    scale = gamma * lax.rsqrt(var + EPS)
    shift = beta - mean * scale
    return scale, shift


def _d1_body(x_ref, w_ref, b_ref, pts_ref, s1_ref, m1_ref):
    x = x_ref[...]
    p = jnp.dot(x, w_ref[...], preferred_element_type=F32) + b_ref[...]
    pts_ref[...] = jnp.concatenate([p, jnp.zeros((RT, CP - C), F32)], axis=1)

    @pl.when(pl.program_id(0) == 0)
    def _():
        s1_ref[...] = jnp.zeros_like(s1_ref)
        m1_ref[...] = jnp.zeros_like(m1_ref)

    s1_ref[...] += jnp.sum(p, axis=0, keepdims=True)
    m1_ref[...] += lax.dot_general(p, p, (((0,), (0,)), ((), ())),
                                   preferred_element_type=F32)


def _d1_call(points_flat, W_fc1, b_fc1):
    return pl.pallas_call(
        _d1_body,
        grid=(NT,),
        in_specs=[
            pl.BlockSpec((RT, C), lambda t: (t, 0)),
            pl.BlockSpec((C, C), lambda t: (0, 0)),
            pl.BlockSpec((1, C), lambda t: (0, 0)),
        ],
        out_specs=[
            pl.BlockSpec((RT, CP), lambda t: (t, 0)),
            pl.BlockSpec((1, C), lambda t: (0, 0)),
            pl.BlockSpec((C, C), lambda t: (0, 0)),
        ],
        out_shape=[
            jax.ShapeDtypeStruct((L_TOT, CP), F32),
            jax.ShapeDtypeStruct((1, C), F32),
            jax.ShapeDtypeStruct((C, C), F32),
        ],
    )(points_flat, W_fc1, b_fc1)


def _d2_body(p_ref, w1_ref, b1_ref, g1_ref, be1_ref, s1_ref, m1_ref,
             s2_ref, m2_ref):
    sc1, sh1 = _bn_scale_shift(w1_ref[...], b1_ref[...], g1_ref[...],
                               be1_ref[...], s1_ref[...], m1_ref[...])
    p = p_ref[:, :C]
    y1 = lax.dot_general(p, w1_ref[...], (((1,), (1,)), ((), ())),
                         preferred_element_type=F32)
    h1 = jnp.maximum(y1 * sc1 + (b1_ref[...] * sc1 + sh1), 0.0)

    @pl.when(pl.program_id(0) == 0)
    def _():
        s2_ref[...] = jnp.zeros_like(s2_ref)
        m2_ref[...] = jnp.zeros_like(m2_ref)

    s2_ref[...] += jnp.sum(h1, axis=0, keepdims=True)
    m2_ref[...] += lax.dot_general(h1, h1, (((0,), (0,)), ((), ())),
                                   preferred_element_type=F32)


def _d2_call(pts, W_c1, b_c1, g1, be1, S1, M1):
    return pl.pallas_call(
        _d2_body,
        grid=(NT,),
        in_specs=[
            pl.BlockSpec((RT, CP), lambda t: (t, 0)),
            pl.BlockSpec((C, C), lambda t: (0, 0)),
            pl.BlockSpec((1, C), lambda t: (0, 0)),
            pl.BlockSpec((1, C), lambda t: (0, 0)),
            pl.BlockSpec((1, C), lambda t: (0, 0)),
            pl.BlockSpec((1, C), lambda t: (0, 0)),
            pl.BlockSpec((C, C), lambda t: (0, 0)),
        ],
        out_specs=[
            pl.BlockSpec((1, C), lambda t: (0, 0)),
            pl.BlockSpec((C, C), lambda t: (0, 0)),
        ],
        out_shape=[
            jax.ShapeDtypeStruct((1, C), F32),
            jax.ShapeDtypeStruct((C, C), F32),
        ],
    )(pts, W_c1, b_c1, g1, be1, S1, M1)


def _d3_body(p_ref, w1_ref, b1_ref, g1_ref, be1_ref, w2_ref, b2_ref, g2_ref,
             be2_ref, s1_ref, m1_ref, s2_ref, m2_ref, out_ref):
    sc1, sh1 = _bn_scale_shift(w1_ref[...], b1_ref[...], g1_ref[...],
                               be1_ref[...], s1_ref[...], m1_ref[...])
    sc2, sh2 = _bn_scale_shift(w2_ref[...], b2_ref[...], g2_ref[...],
                               be2_ref[...], s2_ref[...], m2_ref[...])
    p = p_ref[:, :C]
    y1 = lax.dot_general(p, w1_ref[...], (((1,), (1,)), ((), ())),
                         preferred_element_type=F32)
    h1 = jnp.maximum(y1 * sc1 + (b1_ref[...] * sc1 + sh1), 0.0)
    y2 = lax.dot_general(h1, w2_ref[...], (((1,), (1,)), ((), ())),
                         preferred_element_type=F32)
    h2 = jnp.maximum(y2 * sc2 + (b2_ref[...] * sc2 + sh2), 0.0)
    out_ref[...] = jnp.concatenate([p + h2, jnp.zeros((RT, CP - C), F32)],
                                   axis=1)


def _d3_call(pts, W_c1, b_c1, g1, be1, W_c2, b_c2, g2, be2, S1, M1, S2, M2):
    vec = pl.BlockSpec((1, C), lambda t: (0, 0))
    mat = pl.BlockSpec((C, C), lambda t: (0, 0))
    return pl.pallas_call(
        _d3_body,
        grid=(NT,),
        in_specs=[pl.BlockSpec((RT, CP), lambda t: (t, 0)),
                  mat, vec, vec, vec, mat, vec, vec, vec, vec, mat, vec, mat],
        out_specs=pl.BlockSpec((RT, CP), lambda t: (t, 0)),
        out_shape=jax.ShapeDtypeStruct((L_TOT, CP), F32),
    )(pts, W_c1, b_c1, g1, be1, W_c2, b_c2, g2, be2, S1, M1, S2, M2)


RS = 256
NQT = S // RS


def _knn_body(nx_ref, xp_ref, kidx_ref, d_ref):
    q = nx_ref[0]
    x = xp_ref[0]
    t = lax.dot_general(q, x, (((0,), (0,)), ((), ())),
                        preferred_element_type=F32)
    qsq = jnp.sum(q * q, axis=0)[:, None]
    xsq = jnp.sum(x * x, axis=0)[None, :]
    d_ref[...] = (-2.0 * t + qsq) + xsq

    col = lax.broadcasted_iota(I32, (RS, N), 1)
    kcol = lax.broadcasted_iota(I32, (RS, K), 1)
    big = jnp.float32(3.0e38)

    def step(k, acc):
        d = d_ref[...]
        m = jnp.min(d, axis=1, keepdims=True)
        eq = d == m
        j = jnp.min(jnp.where(eq, col, N), axis=1, keepdims=True)
        acc = jnp.where(kcol == k, j, acc)
        d_ref[...] = jnp.where(eq, big, d)
        return acc

    acc = lax.fori_loop(0, K, step, jnp.zeros((RS, K), I32))
    kidx_ref[0] = acc + pl.program_id(0) * N


def _knn_call(nx_planes, xyz_planes):
    return pl.pallas_call(
        _knn_body,
        grid=(B, NQT),
        in_specs=[
            pl.BlockSpec((1, 3, RS), lambda b, t: (b, 0, t)),
            pl.BlockSpec((1, 3, N), lambda b, t: (b, 0, 0)),
        ],
        out_specs=pl.BlockSpec((1, RS, K), lambda b, t: (b, t, 0)),
        out_shape=jax.ShapeDtypeStruct((B, S, K), I32),
        scratch_shapes=[pltpu.VMEM((RS, N), F32)],
    )(nx_planes, xyz_planes)


def _sca_body(pts_hbm, gidx_hbm, pori_hbm, gidx_v, po_v, sem):
    wid = lax.axis_index("s") * NC + lax.axis_index("c")
    base = wid * RW
    pltpu.sync_copy(gidx_hbm.at[pl.ds(base, RW)], gidx_v)
    pltpu.async_copy(pts_hbm.at[gidx_v], po_v, sem).wait()
    pltpu.sync_copy(po_v, pori_hbm.at[pl.ds(base, RW)])


def _scb_body(pts2_hbm, kidx_hbm, maxp_hbm, kidx_v, grp0_v, grp1_v, res_v,
              sem):
    wid = lax.axis_index("s") * NC + lax.axis_index("c")
    base = wid * RW

    pltpu.sync_copy(kidx_hbm.at[pl.ds(base * K, RW * K)], kidx_v)

    def start(r, grp):
        pltpu.make_async_copy(
            pts2_hbm.at[kidx_v.at[pl.ds(r * K, K)]], grp, sem).start()

    def wait(grp):
        pltpu.make_async_copy(
            pts2_hbm.at[kidx_v.at[pl.ds(0, K)]], grp, sem).wait()

    def compute(r, grp):
        for j in range(C // LANES):
            sl = pl.ds(j * LANES, LANES)
            a = grp[0, sl]
            for k in range(1, K):
                a = jnp.maximum(a, grp[k, sl])
            res_v[pl.ds(r * C + j * LANES, LANES)] = a

    start(0, grp0_v)
    start(1, grp1_v)

    def body(r2, carry):
        r = r2 * 2
        wait(grp0_v)
        compute(r, grp0_v)

        @pl.when(r + 2 < RW)
        def _():
            start(r + 2, grp0_v)

        wait(grp1_v)
        compute(r + 1, grp1_v)

        @pl.when(r + 3 < RW)
        def _():
            start(r + 3, grp1_v)

        return carry

    lax.fori_loop(0, RW // 2, body, 0)
    pltpu.sync_copy(res_v, maxp_hbm.at[pl.ds(base * C, RW * C)])


@functools.lru_cache(maxsize=None)
def _get_sc_kernels():
    mesh = plsc.VectorSubcoreMesh(core_axis_name="c", subcore_axis_name="s",
                                  num_cores=NC, num_subcores=NS)
    sca = functools.partial(
        pl.kernel,
        out_type=jax.ShapeDtypeStruct((B * S, CP), F32),
        mesh=mesh,
        scratch_types=[
            pltpu.VMEM((RW,), I32),
            pltpu.VMEM((RW, CP), F32),
            pltpu.SemaphoreType.DMA,
        ],
    )(_sca_body)
    scb = functools.partial(
        pl.kernel,
        out_type=jax.ShapeDtypeStruct((B * S * C,), F32),
        mesh=mesh,
        scratch_types=[
            pltpu.VMEM((RW * K,), I32),
            pltpu.VMEM((K, CP), F32),
            pltpu.VMEM((K, CP), F32),
            pltpu.VMEM((RW * C,), F32),
            pltpu.SemaphoreType.DMA,
        ],
    )(_scb_body)
    return sca, scb


FT = 2048
NFT = (B * S) // FT


def _fbn_stats_body(mx_ref, po_ref, np_ref, s_ref, q_ref):
    v = mx_ref[...] + po_ref[:, :C]
    np_ref[...] = v

    @pl.when(pl.program_id(0) == 0)
    def _():
        s_ref[...] = jnp.zeros_like(s_ref)
        q_ref[...] = jnp.zeros_like(q_ref)

    s_ref[...] += jnp.sum(v, axis=0, keepdims=True)
    q_ref[...] += jnp.sum(v * v, axis=0, keepdims=True)


def _fbn_stats_call(maxp, pori):
    return pl.pallas_call(
        _fbn_stats_body,
        grid=(NFT,),
        in_specs=[
            pl.BlockSpec((FT, C), lambda t: (t, 0)),
            pl.BlockSpec((FT, CP), lambda t: (t, 0)),
        ],
        out_specs=[
            pl.BlockSpec((FT, C), lambda t: (t, 0)),
            pl.BlockSpec((1, C), lambda t: (0, 0)),
            pl.BlockSpec((1, C), lambda t: (0, 0)),
        ],
        out_shape=[
            jax.ShapeDtypeStruct((B * S, C), F32),
            jax.ShapeDtypeStruct((1, C), F32),
            jax.ShapeDtypeStruct((1, C), F32),
        ],
    )(maxp, pori)


def _fbn_norm_body(np_ref, s_ref, q_ref, g_ref, be_ref, out_ref):
    inv_l = 1.0 / (B * S)
    mean = s_ref[...] * inv_l
    var = q_ref[...] * inv_l - mean * mean
    scale = g_ref[...] * lax.rsqrt(var + EPS)
    shift = be_ref[...] - mean * scale
    out_ref[...] = np_ref[...] * scale + shift


def _fbn_norm_call(newp, ssum, qsum, g_bn, be_bn):
    vec = pl.BlockSpec((1, C), lambda t: (0, 0))
    return pl.pallas_call(
        _fbn_norm_body,
        grid=(NFT,),
        in_specs=[pl.BlockSpec((FT, C), lambda t: (t, 0)), vec, vec, vec, vec],
        out_specs=pl.BlockSpec((FT, C), lambda t: (t, 0)),
        out_shape=jax.ShapeDtypeStruct((B * S, C), F32),
    )(newp, ssum, qsum, g_bn, be_bn)


def kernel(xyz, points, W_fc1, b_fc1, W_c1, b_c1, W_c2, b_c2,
           g_bn1, be_bn1, g_bn2, be_bn2, g_bn, be_bn):
    xyzp = xyz.transpose(0, 2, 1)
    xyzp4 = xyzp.reshape(B, 3, 64, 128)

    b_fc1r = b_fc1.reshape(1, C)
    b1 = b_c1.reshape(1, C)
    b2 = b_c2.reshape(1, C)
    g1 = g_bn1.reshape(1, C)
    be1 = be_bn1.reshape(1, C)
    g2 = g_bn2.reshape(1, C)
    be2 = be_bn2.reshape(1, C)
    gf = g_bn.reshape(1, C)
    bef = be_bn.reshape(1, C)

    gidx4, nxp4 = _fps_call(xyzp4, xyz)
    gidx = gidx4.reshape(B * S)
    nx_planes = nxp4.reshape(B, 3, S)
    new_xyz = nx_planes.transpose(0, 2, 1)

    pts, S1, M1 = _d1_call(points.reshape(L_TOT, C), W_fc1, b_fc1r)

    sca, scb = _get_sc_kernels()
    pori = sca(pts, gidx)

    S2, M2 = _d2_call(pts, W_c1, b1, g1, be1, S1, M1)
    pts2 = _d3_call(pts, W_c1, b1, g1, be1, W_c2, b2, g2, be2, S1, M1, S2, M2)

    kidx = _knn_call(nx_planes, xyzp).reshape(B * S * K)

    maxp = scb(pts2, kidx).reshape(B * S, C)

    newp, ssum, qsum = _fbn_stats_call(maxp, pori)
    new_points = _fbn_norm_call(newp, ssum, qsum, gf, bef).reshape(B, S, C)

    return (new_xyz, new_points)


# speedup vs baseline: 11.8831x; 1.3780x over previous
import functools

import jax
import jax.numpy as jnp
from jax import lax
from jax.experimental import pallas as pl
from jax.experimental.pallas import tpu as pltpu
from jax.experimental.pallas import tpu_sc as plsc

B = 4
N = 8192
S = 2048
K = 32
C = 64
CP = 128
EPS = 1e-5
L_TOT = B * N

NC = 2
NS = 16
NW = NC * NS
LANES = 16
RW = (B * S) // NW

F32 = jnp.float32
I32 = jnp.int32


def _fps_argmax5(v, i, x, y, z):
    t = (v, i, x, y, z)

    def comb(hi, lo):
        cond = (hi[0] > lo[0]) | ((hi[0] == lo[0]) & (hi[1] < lo[1]))
        return tuple(jnp.where(cond, a, b) for a, b in zip(hi, lo))

    r = 64
    while r > 1:
        h = r // 2
        t = comb(tuple(a[:h] for a in t), tuple(a[h:] for a in t))
        r = h
    c = 128
    while c > 1:
        h = c // 2
        t = comb(tuple(a[:, :h] for a in t), tuple(a[:, h:] for a in t))
        c = h
    return t


def _fps_body(xyz_ref, idx_ref, nx_ref):
    row_i = lax.broadcasted_iota(I32, (64, 128), 0)
    col_i = lax.broadcasted_iota(I32, (64, 128), 1)
    flat = row_i * 128 + col_i
    srow = lax.broadcasted_iota(I32, (16, 128), 0)
    scol = lax.broadcasted_iota(I32, (16, 128), 1)
    sflat = srow * 128 + scol

    def body(i, state):
        dists, fs = state
        sel = sflat == i
        new_d = ()
        new_f = ()
        for b in range(B):
            f, cx, cy, cz = fs[b]
            idx_ref[b] = jnp.where(sel, f + b * N, idx_ref[b])
            nx_ref[b, 0] = jnp.where(sel, cx, nx_ref[b, 0])
            nx_ref[b, 1] = jnp.where(sel, cy, nx_ref[b, 1])
            nx_ref[b, 2] = jnp.where(sel, cz, nx_ref[b, 2])
            X = xyz_ref[b, 0]
            Y = xyz_ref[b, 1]
            Z = xyz_ref[b, 2]
            dx = X - cx
            dy = Y - cy
            dz = Z - cz
            d = dx * dx + dy * dy + dz * dz
            dist = jnp.minimum(dists[b], d)
            _, f2, nx2, ny2, nz2 = _fps_argmax5(dist, flat, X, Y, Z)
            new_d += (dist,)
            new_f += ((f2, nx2, ny2, nz2),)
        return new_d, new_f

    dist0 = jnp.full((64, 128), 1e10, F32)
    f0 = jnp.zeros((1, 1), I32)
    c0s = []
    for b in range(B):
        c0s.append((f0, xyz_ref[b, 0, 0:1, 0:1], xyz_ref[b, 1, 0:1, 0:1],
                    xyz_ref[b, 2, 0:1, 0:1]))
    lax.fori_loop(0, S, body, ((dist0,) * B, tuple(c0s)))


def _fps_call(xyz_planes):
    return pl.pallas_call(
        _fps_body,
        in_specs=[pl.BlockSpec((B, 3, 64, 128), lambda: (0, 0, 0, 0))],
        out_specs=[
            pl.BlockSpec((B, 16, 128), lambda: (0, 0, 0)),
            pl.BlockSpec((B, 3, 16, 128), lambda: (0, 0, 0, 0)),
        ],
        out_shape=[
            jax.ShapeDtypeStruct((B, 16, 128), I32),
            jax.ShapeDtypeStruct((B, 3, 16, 128), F32),
        ],
    )(xyz_planes)


RT = 2048
NT = L_TOT // RT


def _bn_scale_shift(Wm, bias, gamma, beta, s_in, m_in):
    inv_l = 1.0 / L_TOT
    ewx = lax.dot_general(s_in, Wm, (((1,), (1,)), ((), ())),
                          preferred_element_type=F32) * inv_l
    mean = ewx + bias
    wm = jnp.dot(Wm, m_in, preferred_element_type=F32)
    ey2 = jnp.sum(wm * Wm, axis=1)[None, :] * inv_l
    ey2 = ey2 + 2.0 * bias * ewx + bias * bias
    var = ey2 - mean * mean
    scale = gamma * lax.rsqrt(var + EPS)
    shift = beta - mean * scale
    return scale, shift


def _d1_body(x_ref, w_ref, b_ref, pts_ref, s1_ref, m1_ref):
    x = x_ref[...]
    p = jnp.dot(x, w_ref[...], preferred_element_type=F32) + b_ref[...]
    pts_ref[...] = jnp.concatenate([p, jnp.zeros((RT, CP - C), F32)], axis=1)

    @pl.when(pl.program_id(0) == 0)
    def _():
        s1_ref[...] = jnp.zeros_like(s1_ref)
        m1_ref[...] = jnp.zeros_like(m1_ref)

    s1_ref[...] += jnp.sum(p, axis=0, keepdims=True)
    m1_ref[...] += lax.dot_general(p, p, (((0,), (0,)), ((), ())),
                                   preferred_element_type=F32)


def _d1_call(points_flat, W_fc1, b_fc1):
    return pl.pallas_call(
        _d1_body,
        grid=(NT,),
        in_specs=[
            pl.BlockSpec((RT, C), lambda t: (t, 0)),
            pl.BlockSpec((C, C), lambda t: (0, 0)),
            pl.BlockSpec((1, C), lambda t: (0, 0)),
        ],
        out_specs=[
            pl.BlockSpec((RT, CP), lambda t: (t, 0)),
            pl.BlockSpec((1, C), lambda t: (0, 0)),
            pl.BlockSpec((C, C), lambda t: (0, 0)),
        ],
        out_shape=[
            jax.ShapeDtypeStruct((L_TOT, CP), F32),
            jax.ShapeDtypeStruct((1, C), F32),
            jax.ShapeDtypeStruct((C, C), F32),
        ],
    )(points_flat, W_fc1, b_fc1)


def _d2_body(p_ref, w1_ref, b1_ref, g1_ref, be1_ref, s1_ref, m1_ref,
             s2_ref, m2_ref):
    sc1, sh1 = _bn_scale_shift(w1_ref[...], b1_ref[...], g1_ref[...],
                               be1_ref[...], s1_ref[...], m1_ref[...])
    p = p_ref[:, :C]
    y1 = lax.dot_general(p, w1_ref[...], (((1,), (1,)), ((), ())),
                         preferred_element_type=F32)
    h1 = jnp.maximum(y1 * sc1 + (b1_ref[...] * sc1 + sh1), 0.0)

    @pl.when(pl.program_id(0) == 0)
    def _():
        s2_ref[...] = jnp.zeros_like(s2_ref)
        m2_ref[...] = jnp.zeros_like(m2_ref)

    s2_ref[...] += jnp.sum(h1, axis=0, keepdims=True)
    m2_ref[...] += lax.dot_general(h1, h1, (((0,), (0,)), ((), ())),
                                   preferred_element_type=F32)


def _d2_call(pts, W_c1, b_c1, g1, be1, S1, M1):
    return pl.pallas_call(
        _d2_body,
        grid=(NT,),
        in_specs=[
            pl.BlockSpec((RT, CP), lambda t: (t, 0)),
            pl.BlockSpec((C, C), lambda t: (0, 0)),
            pl.BlockSpec((1, C), lambda t: (0, 0)),
            pl.BlockSpec((1, C), lambda t: (0, 0)),
            pl.BlockSpec((1, C), lambda t: (0, 0)),
            pl.BlockSpec((1, C), lambda t: (0, 0)),
            pl.BlockSpec((C, C), lambda t: (0, 0)),
        ],
        out_specs=[
            pl.BlockSpec((1, C), lambda t: (0, 0)),
            pl.BlockSpec((C, C), lambda t: (0, 0)),
        ],
        out_shape=[
            jax.ShapeDtypeStruct((1, C), F32),
            jax.ShapeDtypeStruct((C, C), F32),
        ],
    )(pts, W_c1, b_c1, g1, be1, S1, M1)


def _d3_body(p_ref, w1_ref, b1_ref, g1_ref, be1_ref, w2_ref, b2_ref, g2_ref,
             be2_ref, s1_ref, m1_ref, s2_ref, m2_ref, out_ref):
    sc1, sh1 = _bn_scale_shift(w1_ref[...], b1_ref[...], g1_ref[...],
                               be1_ref[...], s1_ref[...], m1_ref[...])
    sc2, sh2 = _bn_scale_shift(w2_ref[...], b2_ref[...], g2_ref[...],
                               be2_ref[...], s2_ref[...], m2_ref[...])
    p = p_ref[:, :C]
    y1 = lax.dot_general(p, w1_ref[...], (((1,), (1,)), ((), ())),
                         preferred_element_type=F32)
    h1 = jnp.maximum(y1 * sc1 + (b1_ref[...] * sc1 + sh1), 0.0)
    y2 = lax.dot_general(h1, w2_ref[...], (((1,), (1,)), ((), ())),
                         preferred_element_type=F32)
    h2 = jnp.maximum(y2 * sc2 + (b2_ref[...] * sc2 + sh2), 0.0)
    out_ref[...] = jnp.concatenate([p + h2, jnp.zeros((RT, CP - C), F32)],
                                   axis=1)


def _d3_call(pts, W_c1, b_c1, g1, be1, W_c2, b_c2, g2, be2, S1, M1, S2, M2):
    vec = pl.BlockSpec((1, C), lambda t: (0, 0))
    mat = pl.BlockSpec((C, C), lambda t: (0, 0))
    return pl.pallas_call(
        _d3_body,
        grid=(NT,),
        in_specs=[pl.BlockSpec((RT, CP), lambda t: (t, 0)),
                  mat, vec, vec, vec, mat, vec, vec, vec, vec, mat, vec, mat],
        out_specs=pl.BlockSpec((RT, CP), lambda t: (t, 0)),
        out_shape=jax.ShapeDtypeStruct((L_TOT, CP), F32),
    )(pts, W_c1, b_c1, g1, be1, W_c2, b_c2, g2, be2, S1, M1, S2, M2)


RS = 256
NQT = S // RS


def _knn_body(nx_ref, xp_ref, kidx_ref, d_ref):
    q = nx_ref[0]
    x = xp_ref[0]
    t = lax.dot_general(q, x, (((0,), (0,)), ((), ())),
                        preferred_element_type=F32)
    qsq = jnp.sum(q * q, axis=0)[:, None]
    xsq = jnp.sum(x * x, axis=0)[None, :]
    d_ref[...] = (-2.0 * t + qsq) + xsq

    col = lax.broadcasted_iota(I32, (RS, N), 1)
    kcol = lax.broadcasted_iota(I32, (RS, K), 1)
    big = jnp.float32(3.0e38)

    def step(k, acc):
        d = d_ref[...]
        m = jnp.min(d, axis=1, keepdims=True)
        eq = d == m
        j = jnp.min(jnp.where(eq, col, N), axis=1, keepdims=True)
        acc = jnp.where(kcol == k, j, acc)
        d_ref[...] = jnp.where(eq, big, d)
        return acc

    acc = lax.fori_loop(0, K, step, jnp.zeros((RS, K), I32))
    kidx_ref[0] = acc + pl.program_id(0) * N


def _knn_call(nx_planes, xyz_planes):
    return pl.pallas_call(
        _knn_body,
        grid=(B, NQT),
        in_specs=[
            pl.BlockSpec((1, 3, RS), lambda b, t: (b, 0, t)),
            pl.BlockSpec((1, 3, N), lambda b, t: (b, 0, 0)),
        ],
        out_specs=pl.BlockSpec((1, RS, K), lambda b, t: (b, t, 0)),
        out_shape=jax.ShapeDtypeStruct((B, S, K), I32),
        scratch_shapes=[pltpu.VMEM((RS, N), F32)],
    )(nx_planes, xyz_planes)


def _sca_body(pts_hbm, gidx_hbm, pori_hbm, gidx_v, po_v, sem):
    wid = lax.axis_index("s") * NC + lax.axis_index("c")
    base = wid * RW
    pltpu.sync_copy(gidx_hbm.at[pl.ds(base, RW)], gidx_v)
    pltpu.async_copy(pts_hbm.at[gidx_v], po_v, sem).wait()
    pltpu.sync_copy(po_v, pori_hbm.at[pl.ds(base, RW)])


def _scb_body(pts2_hbm, kidx_hbm, maxp_hbm, kidx_v, grp0_v, grp1_v, res_v,
              sem):
    wid = lax.axis_index("s") * NC + lax.axis_index("c")
    base = wid * RW

    pltpu.sync_copy(kidx_hbm.at[pl.ds(base * K, RW * K)], kidx_v)

    def start(r, grp):
        pltpu.make_async_copy(
            pts2_hbm.at[kidx_v.at[pl.ds(r * K, K)]], grp, sem).start()

    def wait(grp):
        pltpu.make_async_copy(
            pts2_hbm.at[kidx_v.at[pl.ds(0, K)]], grp, sem).wait()

    def compute(r, grp):
        for j in range(C // LANES):
            sl = pl.ds(j * LANES, LANES)
            a = grp[0, sl]
            for k in range(1, K):
                a = jnp.maximum(a, grp[k, sl])
            res_v[pl.ds(r * C + j * LANES, LANES)] = a

    start(0, grp0_v)
    start(1, grp1_v)

    def body(r2, carry):
        r = r2 * 2
        wait(grp0_v)
        compute(r, grp0_v)

        @pl.when(r + 2 < RW)
        def _():
            start(r + 2, grp0_v)

        wait(grp1_v)
        compute(r + 1, grp1_v)

        @pl.when(r + 3 < RW)
        def _():
            start(r + 3, grp1_v)

        return carry

    lax.fori_loop(0, RW // 2, body, 0)
    pltpu.sync_copy(res_v, maxp_hbm.at[pl.ds(base * C, RW * C)])


@functools.lru_cache(maxsize=None)
def _get_sc_kernels():
    mesh = plsc.VectorSubcoreMesh(core_axis_name="c", subcore_axis_name="s",
                                  num_cores=NC, num_subcores=NS)
    sca = functools.partial(
        pl.kernel,
        out_type=jax.ShapeDtypeStruct((B * S, CP), F32),
        mesh=mesh,
        scratch_types=[
            pltpu.VMEM((RW,), I32),
            pltpu.VMEM((RW, CP), F32),
            pltpu.SemaphoreType.DMA,
        ],
    )(_sca_body)
    scb = functools.partial(
        pl.kernel,
        out_type=jax.ShapeDtypeStruct((B * S * C,), F32),
        mesh=mesh,
        scratch_types=[
            pltpu.VMEM((RW * K,), I32),
            pltpu.VMEM((K, CP), F32),
            pltpu.VMEM((K, CP), F32),
            pltpu.VMEM((RW * C,), F32),
            pltpu.SemaphoreType.DMA,
        ],
    )(_scb_body)
    return sca, scb


FT = 2048
NFT = (B * S) // FT


def _fbn_stats_body(mx_ref, po_ref, np_ref, s_ref, q_ref):
    v = mx_ref[...] + po_ref[:, :C]
    np_ref[...] = v

    @pl.when(pl.program_id(0) == 0)
    def _():
        s_ref[...] = jnp.zeros_like(s_ref)
        q_ref[...] = jnp.zeros_like(q_ref)

    s_ref[...] += jnp.sum(v, axis=0, keepdims=True)
    q_ref[...] += jnp.sum(v * v, axis=0, keepdims=True)


def _fbn_stats_call(maxp, pori):
    return pl.pallas_call(
        _fbn_stats_body,
        grid=(NFT,),
        in_specs=[
            pl.BlockSpec((FT, C), lambda t: (t, 0)),
            pl.BlockSpec((FT, CP), lambda t: (t, 0)),
        ],
        out_specs=[
            pl.BlockSpec((FT, C), lambda t: (t, 0)),
            pl.BlockSpec((1, C), lambda t: (0, 0)),
            pl.BlockSpec((1, C), lambda t: (0, 0)),
        ],
        out_shape=[
            jax.ShapeDtypeStruct((B * S, C), F32),
            jax.ShapeDtypeStruct((1, C), F32),
            jax.ShapeDtypeStruct((1, C), F32),
        ],
    )(maxp, pori)


def _fbn_norm_body(np_ref, s_ref, q_ref, g_ref, be_ref, out_ref):
    inv_l = 1.0 / (B * S)
    mean = s_ref[...] * inv_l
    var = q_ref[...] * inv_l - mean * mean
    scale = g_ref[...] * lax.rsqrt(var + EPS)
    shift = be_ref[...] - mean * scale
    out_ref[...] = np_ref[...] * scale + shift


def _fbn_norm_call(newp, ssum, qsum, g_bn, be_bn):
    vec = pl.BlockSpec((1, C), lambda t: (0, 0))
    return pl.pallas_call(
        _fbn_norm_body,
        grid=(NFT,),
        in_specs=[pl.BlockSpec((FT, C), lambda t: (t, 0)), vec, vec, vec, vec],
        out_specs=pl.BlockSpec((FT, C), lambda t: (t, 0)),
        out_shape=jax.ShapeDtypeStruct((B * S, C), F32),
    )(newp, ssum, qsum, g_bn, be_bn)


def kernel(xyz, points, W_fc1, b_fc1, W_c1, b_c1, W_c2, b_c2,
           g_bn1, be_bn1, g_bn2, be_bn2, g_bn, be_bn):
    xyzp = xyz.transpose(0, 2, 1)
    xyzp4 = xyzp.reshape(B, 3, 64, 128)

    b_fc1r = b_fc1.reshape(1, C)
    b1 = b_c1.reshape(1, C)
    b2 = b_c2.reshape(1, C)
    g1 = g_bn1.reshape(1, C)
    be1 = be_bn1.reshape(1, C)
    g2 = g_bn2.reshape(1, C)
    be2 = be_bn2.reshape(1, C)
    gf = g_bn.reshape(1, C)
    bef = be_bn.reshape(1, C)

    gidx4, nxp4 = _fps_call(xyzp4)
    gidx = gidx4.reshape(B * S)
    nx_planes = nxp4.reshape(B, 3, S)
    new_xyz = nx_planes.transpose(0, 2, 1)

    pts, S1, M1 = _d1_call(points.reshape(L_TOT, C), W_fc1, b_fc1r)

    sca, scb = _get_sc_kernels()
    pori = sca(pts, gidx)

    S2, M2 = _d2_call(pts, W_c1, b1, g1, be1, S1, M1)
    pts2 = _d3_call(pts, W_c1, b1, g1, be1, W_c2, b2, g2, be2, S1, M1, S2, M2)

    kidx = _knn_call(nx_planes, xyzp).reshape(B * S * K)

    maxp = scb(pts2, kidx).reshape(B * S, C)

    newp, ssum, qsum = _fbn_stats_call(maxp, pori)
    new_points = _fbn_norm_call(newp, ssum, qsum, gf, bef).reshape(B, S, C)

    return (new_xyz, new_points)
